# baseline (speedup 1.0000x reference)
import functools

import jax
import jax.numpy as jnp
from jax import lax
from jax.experimental import pallas as pl
from jax.experimental.pallas import tpu as pltpu
from jax.experimental.pallas import tpu_sc as plsc

_DIM = 256
_NE = 1024
_BLK = 2048
_N_TOK = 8192

_NC = 2
_NS = 16
_NW = _NC * _NS
_BPW = _N_TOK // _NW


def _vq_tc_kernel(x_ref, e_ref, idx_ref, diff_ref):
    x = x_ref[...]
    e = e_ref[...]
    xsq = jnp.sum(x * x, axis=1, keepdims=True)
    esq = jnp.sum(e * e, axis=0, keepdims=True)
    xe = jnp.dot(x, e, preferred_element_type=jnp.float32)
    dist = xsq - 2.0 * xe + esq
    minv = jnp.min(dist, axis=1, keepdims=True)
    lane_f = jax.lax.broadcasted_iota(
        jnp.int32, (_BLK, _NE), 1).astype(jnp.float32)
    idx_f = jnp.min(jnp.where(dist == minv, lane_f, jnp.float32(_NE)),
                    axis=1, keepdims=True)
    idx_ref[...] = idx_f.astype(jnp.int32)

    d = jnp.sum(minv).reshape(1, 1)

    @pl.when(pl.program_id(0) == 0)
    def _():
        diff_ref[...] = jnp.zeros((1, 1), jnp.float32)

    diff_ref[...] += d


def _sc_gather_kernel(table_hbm, idx_hbm, out_hbm, idx_v, rows_v, sem):
    wid = lax.axis_index("s") * _NC + lax.axis_index("c")
    base = wid * _BPW
    pltpu.sync_copy(idx_hbm.at[pl.ds(base, _BPW)], idx_v)
    pltpu.async_copy(table_hbm.at[idx_v], rows_v, sem).wait()
    pltpu.sync_copy(rows_v, out_hbm.at[pl.ds(base, _BPW)])


def kernel(input, embed):
    flat = input.reshape(-1, _DIM)
    n_tok = flat.shape[0]
    nblk = n_tok // _BLK
    idx2, diff = pl.pallas_call(
        _vq_tc_kernel,
        grid=(nblk,),
        in_specs=[
            pl.BlockSpec((_BLK, _DIM), lambda i: (i, 0)),
            pl.BlockSpec((_DIM, _NE), lambda i: (0, 0)),
        ],
        out_specs=[
            pl.BlockSpec((_BLK, 1), lambda i: (i, 0)),
            pl.BlockSpec((1, 1), lambda i: (0, 0)),
        ],
        out_shape=[
            jax.ShapeDtypeStruct((n_tok, 1), jnp.int32),
            jax.ShapeDtypeStruct((1, 1), jnp.float32),
        ],
    )(flat, embed)

    embed_t = jnp.swapaxes(embed, 0, 1)
    gather = pl.kernel(
        _sc_gather_kernel,
        mesh=plsc.VectorSubcoreMesh(core_axis_name="c", subcore_axis_name="s"),
        out_type=jax.ShapeDtypeStruct((n_tok, _DIM), jnp.float32),
        scratch_types=[
            pltpu.VMEM((_BPW,), jnp.int32),
            pltpu.VMEM((_BPW, _DIM), jnp.float32),
            pltpu.SemaphoreType.DMA,
        ],
    )
    q = gather(embed_t, idx2.reshape(n_tok))

    quantize = q.reshape(input.shape)
    embed_ind = idx2.reshape(input.shape[:-1])
    diff_scalar = diff[0, 0] / jnp.float32(n_tok * _DIM)
    return (quantize, diff_scalar, embed_ind)


# speedup vs baseline: 1.7538x; 1.7538x over previous
import functools

import jax
import jax.numpy as jnp
from jax.experimental import pallas as pl

_DIM = 256
_NE = 1024
_BLK = 2048


def _vq_kernel(x_ref, e_ref, ec_ref, q_ref, idx_ref, diff_ref):
    x = x_ref[...]
    e = e_ref[...]
    xsq = jnp.sum(x * x, axis=1, keepdims=True)
    esq = jnp.sum(e * e, axis=0, keepdims=True)
    xe = jnp.dot(x, e, preferred_element_type=jnp.float32)
    dist = xsq - 2.0 * xe + esq
    minv = jnp.min(dist, axis=1, keepdims=True)
    lane_f = jax.lax.broadcasted_iota(
        jnp.int32, (_BLK, _NE), 1).astype(jnp.float32)
    idx_f = jnp.min(jnp.where(dist == minv, lane_f, jnp.float32(_NE)),
                    axis=1, keepdims=True)
    idx_ref[...] = idx_f.astype(jnp.int32)

    onehot = (lane_f == idx_f).astype(jnp.bfloat16)
    dims = (((1,), (1,)), ((), ()))
    qcat = jax.lax.dot_general(onehot, ec_ref[...], dims,
                               preferred_element_type=jnp.float32)
    q_ref[...] = qcat[:, :_DIM] + qcat[:, _DIM:]

    d = jnp.sum(minv).reshape(1, 1)

    @pl.when(pl.program_id(0) == 0)
    def _():
        diff_ref[...] = jnp.zeros((1, 1), jnp.float32)

    diff_ref[...] += d


def kernel(input, embed):
    flat = input.reshape(-1, _DIM)
    n_tok = flat.shape[0]
    nblk = n_tok // _BLK
    e_hi = embed.astype(jnp.bfloat16)
    e_lo = (embed - e_hi.astype(jnp.float32)).astype(jnp.bfloat16)
    e_cat = jnp.concatenate([e_hi, e_lo], axis=0)
    q, idx3, diff = pl.pallas_call(
        _vq_kernel,
        grid=(nblk,),
        in_specs=[
            pl.BlockSpec((_BLK, _DIM), lambda i: (i, 0)),
            pl.BlockSpec((_DIM, _NE), lambda i: (0, 0)),
            pl.BlockSpec((2 * _DIM, _NE), lambda i: (0, 0)),
        ],
        out_specs=[
            pl.BlockSpec((_BLK, _DIM), lambda i: (i, 0)),
            pl.BlockSpec((_BLK, 1), lambda i: (i, 0)),
            pl.BlockSpec((1, 1), lambda i: (0, 0)),
        ],
        out_shape=[
            jax.ShapeDtypeStruct((n_tok, _DIM), jnp.float32),
            jax.ShapeDtypeStruct((n_tok, 1), jnp.int32),
            jax.ShapeDtypeStruct((1, 1), jnp.float32),
        ],
    )(flat, embed, e_cat)
    quantize = q.reshape(input.shape)
    embed_ind = idx3.reshape(input.shape[:-1])
    diff_scalar = diff[0, 0] / jnp.float32(n_tok * _DIM)
    return (quantize, diff_scalar, embed_ind)


# speedup vs baseline: 2.0380x; 1.1620x over previous
import jax
import jax.numpy as jnp
from jax.experimental import pallas as pl

_DIM = 256
_NE = 1024
_BLK = 2048
_SUB = 2
_SB = _BLK // _SUB


def _vq_kernel(x_ref, e_ref, q_ref, idx_ref, diff_ref):
    e = e_ref[...]
    esq = jnp.sum(e * e, axis=0, keepdims=True)
    e_hi = e.astype(jnp.bfloat16)
    e_lo = (e - e_hi.astype(jnp.float32)).astype(jnp.bfloat16)
    lane_f = jax.lax.broadcasted_iota(
        jnp.int32, (_SB, _NE), 1).astype(jnp.float32)
    dims = (((1,), (1,)), ((), ()))

    d_acc = jnp.zeros((1, 1), jnp.float32)
    for s in range(_SUB):
        rows = pl.ds(s * _SB, _SB)
        x = x_ref[rows, :]
        xsq = jnp.sum(x * x, axis=1, keepdims=True)
        xe = jnp.dot(x, e, preferred_element_type=jnp.float32)
        dist = xsq - 2.0 * xe + esq
        minv = jnp.min(dist, axis=1, keepdims=True)
        idx_f = jnp.min(jnp.where(dist == minv, lane_f, jnp.float32(_NE)),
                        axis=1, keepdims=True)
        idx_ref[rows, :] = idx_f.astype(jnp.int32)

        onehot = (lane_f == idx_f).astype(jnp.bfloat16)
        q = (jax.lax.dot_general(onehot, e_hi, dims,
                                 preferred_element_type=jnp.float32)
             + jax.lax.dot_general(onehot, e_lo, dims,
                                   preferred_element_type=jnp.float32))
        q_ref[rows, :] = q

        d_acc = d_acc + jnp.sum(minv).reshape(1, 1)

    @pl.when(pl.program_id(0) == 0)
    def _():
        diff_ref[...] = jnp.zeros((1, 1), jnp.float32)

    diff_ref[...] += d_acc


def kernel(input, embed):
    flat = input.reshape(-1, _DIM)
    n_tok = flat.shape[0]
    nblk = n_tok // _BLK
    q, idx2, diff = pl.pallas_call(
        _vq_kernel,
        grid=(nblk,),
        in_specs=[
            pl.BlockSpec((_BLK, _DIM), lambda i: (i, 0)),
            pl.BlockSpec((_DIM, _NE), lambda i: (0, 0)),
        ],
        out_specs=[
            pl.BlockSpec((_BLK, _DIM), lambda i: (i, 0)),
            pl.BlockSpec((_BLK, 1), lambda i: (i, 0)),
            pl.BlockSpec((1, 1), lambda i: (0, 0)),
        ],
        out_shape=[
            jax.ShapeDtypeStruct((n_tok, _DIM), jnp.float32),
            jax.ShapeDtypeStruct((n_tok, 1), jnp.int32),
            jax.ShapeDtypeStruct((1, 1), jnp.float32),
        ],
    )(flat, embed)
    quantize = q.reshape(input.shape)
    embed_ind = idx2.reshape(input.shape[:-1])
    diff_scalar = diff[0, 0] / jnp.float32(n_tok * _DIM)
    return (quantize, diff_scalar, embed_ind)
